# Initial kernel scaffold; baseline (speedup 1.0000x reference)
import jax
import jax.numpy as jnp
from jax.experimental import pallas as pl


def kernel(x, table, pe):
    raise NotImplementedError("write your pallas kernel here")


# speedup vs baseline: 2.1707x; 2.1707x over previous
import functools

import jax
import jax.numpy as jnp
from jax import lax
from jax.experimental import pallas as pl
from jax.experimental.pallas import tpu as pltpu
from jax.experimental.pallas import tpu_sc as plsc

PAD_ID = 0
_LANES = 16


def _make_sc_kernel(n_flat, seq, d):
    nw = 32
    per_w = n_flat // nw
    ch = 32
    n_ch = per_w // ch
    n_vec = d // _LANES

    mesh = plsc.VectorSubcoreMesh(core_axis_name="c", subcore_axis_name="s")

    @functools.partial(
        pl.kernel,
        mesh=mesh,
        out_type=jax.ShapeDtypeStruct((n_flat, d), jnp.float32),
        scratch_types=[
            pltpu.VMEM((per_w,), jnp.int32),
            pltpu.VMEM((ch, d), jnp.float32),
            pltpu.VMEM((ch, d), jnp.float32),
            pltpu.SemaphoreType.DMA,
        ],
    )
    def emb(x_hbm, table_hbm, pe_hbm, out_hbm, idx_v, tok_v, pe_v, sem):
        cid = lax.axis_index("c")
        sid = lax.axis_index("s")
        wid = sid * 2 + cid
        base = wid * per_w
        s0 = base % seq

        pltpu.sync_copy(x_hbm.at[pl.ds(base, per_w)], idx_v)

        def chunk_body(c, _):
            r0 = c * ch
            pltpu.async_copy(
                table_hbm.at[idx_v.at[pl.ds(r0, ch)]], tok_v, sem
            ).wait()
            pltpu.sync_copy(pe_hbm.at[pl.ds(s0 + r0, ch)], pe_v)

            ms = []
            for g in range(ch // _LANES):
                iv = idx_v[pl.ds(r0 + g * _LANES, _LANES)]
                mv = jnp.where(iv != PAD_ID, 1.0, 0.0)
                for r16 in range(_LANES):
                    ms.append(mv[r16])

            def col_body(j, _):
                o = j * _LANES
                for row in range(ch):
                    t = tok_v[row, pl.ds(o, _LANES)]
                    p = pe_v[row, pl.ds(o, _LANES)]
                    tok_v[row, pl.ds(o, _LANES)] = t * ms[row] + p
                return 0

            lax.fori_loop(0, n_vec, col_body, 0)
            pltpu.sync_copy(tok_v, out_hbm.at[pl.ds(base + r0, ch)])
            return 0

        lax.fori_loop(0, n_ch, chunk_body, 0)

    return emb


@jax.jit
def kernel(x, table, pe):
    b, s = x.shape
    d = table.shape[1]
    xf = x.reshape(b * s).astype(jnp.int32)
    emb = _make_sc_kernel(b * s, s, d)
    out = emb(xf, table, pe[:s])
    return out.reshape(b, s, d)


# speedup vs baseline: 3.2998x; 1.5202x over previous
import functools

import jax
import jax.numpy as jnp
from jax import lax
from jax.experimental import pallas as pl
from jax.experimental.pallas import tpu as pltpu
from jax.experimental.pallas import tpu_sc as plsc

PAD_ID = 0
_LANES = 16


def _make_sc_kernel(n_flat, seq, d):
    nw = 32
    per_w = n_flat // nw
    ch = 32
    n_ch = per_w // ch
    n_vec = d // _LANES

    mesh = plsc.VectorSubcoreMesh(core_axis_name="c", subcore_axis_name="s")

    @functools.partial(
        pl.kernel,
        mesh=mesh,
        out_type=jax.ShapeDtypeStruct((n_flat, d), jnp.float32),
        scratch_types=[
            pltpu.VMEM((per_w,), jnp.int32),
            pltpu.VMEM((ch, d), jnp.float32),
            pltpu.VMEM((ch, d), jnp.float32),
            pltpu.VMEM((ch, d), jnp.float32),
            pltpu.VMEM((ch, d), jnp.float32),
            pltpu.SemaphoreType.DMA,
            pltpu.SemaphoreType.DMA,
            pltpu.SemaphoreType.DMA,
            pltpu.SemaphoreType.DMA,
            pltpu.SemaphoreType.DMA,
            pltpu.SemaphoreType.DMA,
        ],
    )
    def emb(x_hbm, table_hbm, pe_hbm, out_hbm,
            idx_v, tok0, tok1, pe0, pe1, g0, g1, p0, p1, s0_, s1_):
        cid = lax.axis_index("c")
        sid = lax.axis_index("s")
        wid = sid * 2 + cid
        base = wid * per_w
        pe_base = base % seq

        toks = [tok0, tok1]
        pes = [pe0, pe1]
        gsems = [g0, g1]
        psems = [p0, p1]
        ssems = [s0_, s1_]

        pltpu.sync_copy(x_hbm.at[pl.ds(base, per_w)], idx_v)

        gd, pd, sd = {}, {}, {}

        def start(c):
            b = c % 2
            r0 = c * ch
            gd[c] = pltpu.async_copy(
                table_hbm.at[idx_v.at[pl.ds(r0, ch)]], toks[b], gsems[b]
            )
            pd[c] = pltpu.async_copy(
                pe_hbm.at[pl.ds(pe_base + r0, ch)], pes[b], psems[b]
            )

        start(0)
        for c in range(n_ch):
            b = c % 2
            r0 = c * ch
            if c + 1 < n_ch:
                if c >= 1:
                    sd[c - 1].wait()
                start(c + 1)
            gd[c].wait()
            pd[c].wait()

            ms = []
            for g in range(ch // _LANES):
                iv = idx_v[pl.ds(r0 + g * _LANES, _LANES)]
                mv = jnp.where(iv != PAD_ID, 1.0, 0.0)
                for r16 in range(_LANES):
                    ms.append(mv[r16])

            tok_v, pe_v = toks[b], pes[b]

            def col_body(j, _, tok_v=tok_v, pe_v=pe_v, ms=ms):
                o = j * _LANES
                for row in range(ch):
                    t = tok_v[row, pl.ds(o, _LANES)]
                    p = pe_v[row, pl.ds(o, _LANES)]
                    tok_v[row, pl.ds(o, _LANES)] = t * ms[row] + p
                return 0

            lax.fori_loop(0, n_vec, col_body, 0)

            sd[c] = pltpu.async_copy(
                tok_v, out_hbm.at[pl.ds(base + r0, ch)], ssems[b]
            )
        sd[n_ch - 2].wait()
        sd[n_ch - 1].wait()

    return emb


@jax.jit
def kernel(x, table, pe):
    b, s = x.shape
    d = table.shape[1]
    xf = x.reshape(b * s).astype(jnp.int32)
    emb = _make_sc_kernel(b * s, s, d)
    out = emb(xf, table, pe[:s])
    return out.reshape(b, s, d)
